# baseline (speedup 1.0000x reference)
import math

import jax
import jax.numpy as jnp
from jax import lax
from jax.experimental import pallas as pl
from jax.experimental.pallas import tpu as pltpu
from jax.experimental.pallas import tpu_sc as plsc

_V = 1000000
_D = 64
_B = 4096
_H = 200
_NC = 2
_NS = 16
_NW = _NC * _NS
_BB = _B // _NW
_NG = 4
_NO = 2
_SCALE = math.sqrt(_D)
_L = 16
_TCB = 1024


def _relayout_body(lutt_ref, out_ref):
    out_ref[...] = lutt_ref[...].T * _SCALE


def _gather_body(xt_hbm, lut_hbm, out_hbm, idx_all, gbuf, obuf, gsem, ssem):
    wid = lax.axis_index("s") * _NC + lax.axis_index("c")
    b0 = wid * _BB
    lane = lax.iota(jnp.int32, _L)

    pltpu.sync_copy(xt_hbm.at[:, pl.ds(b0, _BB)], idx_all)

    def start_gather(s, h):
        pltpu.async_copy(lut_hbm.at[idx_all.at[h]], gbuf.at[s], gsem.at[s])

    def wait_gather(s):
        pltpu.make_async_copy(lut_hbm.at[idx_all.at[0]], gbuf.at[s],
                              gsem.at[s]).wait()

    def start_store(so, h):
        pltpu.async_copy(obuf.at[so], out_hbm.at[h, :, pl.ds(b0, _BB)],
                         ssem.at[so])

    def wait_store(so):
        pltpu.make_async_copy(obuf.at[so], out_hbm.at[0, :, pl.ds(b0, _BB)],
                              ssem.at[so]).wait()

    for s in range(_NG):
        start_gather(s, s)

    def outer(g, carry):
        for s in range(_NG):
            h = g * _NG + s
            so = s % _NO
            wait_gather(s)

            @pl.when(h >= _NO)
            def _():
                wait_store(so)

            src = gbuf.at[s]
            dst = obuf.at[so]

            @plsc.parallel_loop(0, _L, step=1, unroll=4)
            def _(d):
                diag = (lane + d) & (_L - 1)
                for db in range(_D // _L):
                    for jb in range(_BB // _L):
                        vals = plsc.load_gather(
                            src, [lane + jb * _L, diag + db * _L])
                        plsc.store_scatter(
                            dst, [diag + db * _L, lane + jb * _L], vals)

            start_store(so, h)

            @pl.when(g < (_H // _NG) - 1)
            def _():
                start_gather(s, h + _NG)
        return carry

    lax.fori_loop(0, _H // _NG, outer, 0)

    for so in range(_NO):
        wait_store(so)


def kernel(x, lut):
    xt = x.T.astype(jnp.int32)
    lutt = lut.T
    nblk = pl.cdiv(_V, _TCB)
    lut_rm = pl.pallas_call(
        _relayout_body,
        grid=(nblk,),
        in_specs=[pl.BlockSpec((_D, _TCB), lambda i: (0, i))],
        out_specs=pl.BlockSpec((_TCB, _D), lambda i: (i, 0)),
        out_shape=jax.ShapeDtypeStruct((_V, _D), jnp.float32),
    )(lutt)
    mesh = plsc.VectorSubcoreMesh(
        core_axis_name="c", subcore_axis_name="s",
        num_cores=_NC, num_subcores=_NS,
    )
    params = pltpu.CompilerParams(
        use_tc_tiling_on_sc=False, needs_layout_passes=False,
    )
    out_t = pl.kernel(
        _gather_body,
        out_type=jax.ShapeDtypeStruct((_H, _D, _B), jnp.float32),
        mesh=mesh,
        compiler_params=params,
        scratch_types=[
            pltpu.VMEM((_H, _BB), jnp.int32),
            pltpu.VMEM((_NG, _BB, _D), jnp.float32),
            pltpu.VMEM((_NO, _D, _BB), jnp.float32),
            pltpu.SemaphoreType.DMA((_NG,)),
            pltpu.SemaphoreType.DMA((_NO,)),
        ],
    )(xt, lut_rm)
    return out_t.transpose(2, 0, 1)


# speedup vs baseline: 1.0231x; 1.0231x over previous
import math

import jax
import jax.numpy as jnp
from jax import lax
from jax.experimental import pallas as pl
from jax.experimental.pallas import tpu as pltpu
from jax.experimental.pallas import tpu_sc as plsc

_V = 1000000
_F = 499968
_VP = _F + 64
_D = 64
_B = 4096
_H = 200
_NC = 2
_NS = 16
_NW = _NC * _NS
_BB = _B // _NW
_NG = 4
_NO = 2
_SCALE = math.sqrt(_D)
_L = 16
_W = 256


def _relayout_body(a_ref, b_ref, out_ref):
    out_ref[:, :_D] = a_ref[...].T * _SCALE
    out_ref[:, _D:] = b_ref[...].T * _SCALE


def _gather_body(xt_hbm, lut2_hbm, out_hbm, idx_all, pidx, poff, gbuf, obuf,
                 gsem, ssem):
    wid = lax.axis_index("s") * _NC + lax.axis_index("c")
    b0 = wid * _BB
    lane = lax.iota(jnp.int32, _L)

    pltpu.sync_copy(xt_hbm.at[:, pl.ds(b0, _BB)], idx_all)

    def prep_and_gather(s, h):
        for j in range(_BB // _L):
            sl = pl.ds(j * _L, _L)
            v = idx_all[h, sl]
            big = v >= _F
            pidx[s, sl] = jnp.where(big, v - _F, v)
            poff[s, sl] = jnp.where(big, _D, 0)
        pltpu.async_copy(lut2_hbm.at[pidx.at[s]], gbuf.at[s], gsem.at[s])

    def wait_gather(s):
        pltpu.make_async_copy(lut2_hbm.at[pidx.at[s]], gbuf.at[s],
                              gsem.at[s]).wait()

    def start_store(so, h):
        pltpu.async_copy(obuf.at[so], out_hbm.at[h, :, pl.ds(b0, _BB)],
                         ssem.at[so])

    def wait_store(so):
        pltpu.make_async_copy(obuf.at[so], out_hbm.at[0, :, pl.ds(b0, _BB)],
                              ssem.at[so]).wait()

    for s in range(_NG):
        prep_and_gather(s, s)

    def outer(g, carry):
        for s in range(_NG):
            h = g * _NG + s
            so = s % _NO
            wait_gather(s)

            @pl.when(h >= _NO)
            def _():
                wait_store(so)

            src = gbuf.at[s]
            dst = obuf.at[so]
            offs = [poff[s, pl.ds(jb * _L, _L)] for jb in range(_BB // _L)]

            @plsc.parallel_loop(0, _L, step=1, unroll=4)
            def _(d):
                diag = (lane + d) & (_L - 1)
                for db in range(_D // _L):
                    for jb in range(_BB // _L):
                        vals = plsc.load_gather(
                            src,
                            [lane + jb * _L, offs[jb] + (diag + db * _L)])
                        plsc.store_scatter(
                            dst, [diag + db * _L, lane + jb * _L], vals)

            start_store(so, h)

            @pl.when(g < (_H // _NG) - 1)
            def _():
                prep_and_gather(s, h + _NG)
        return carry

    lax.fori_loop(0, _H // _NG, outer, 0)

    for so in range(_NO):
        wait_store(so)


def kernel(x, lut):
    xt = x.T.astype(jnp.int32)
    lutt = lut.T
    nblk = pl.cdiv(_VP, _W)
    lut2 = pl.pallas_call(
        _relayout_body,
        grid=(nblk,),
        in_specs=[
            pl.BlockSpec((_D, _W), lambda i: (0, i)),
            pl.BlockSpec((_D, _W), lambda i: (0, _F // _W + i)),
        ],
        out_specs=pl.BlockSpec((_W, 2 * _D), lambda i: (i, 0)),
        out_shape=jax.ShapeDtypeStruct((_VP, 2 * _D), jnp.float32),
    )(lutt, lutt)
    mesh = plsc.VectorSubcoreMesh(
        core_axis_name="c", subcore_axis_name="s",
        num_cores=_NC, num_subcores=_NS,
    )
    params = pltpu.CompilerParams(
        use_tc_tiling_on_sc=True, needs_layout_passes=False,
    )
    out_t = pl.kernel(
        _gather_body,
        out_type=jax.ShapeDtypeStruct((_H, _D, _B), jnp.float32),
        mesh=mesh,
        compiler_params=params,
        scratch_types=[
            pltpu.VMEM((_H, _BB), jnp.int32),
            pltpu.VMEM((_NG, _BB), jnp.int32),
            pltpu.VMEM((_NG, _BB), jnp.int32),
            pltpu.VMEM((_NG, _BB, 2 * _D), jnp.float32),
            pltpu.VMEM((_NO, _D, _BB), jnp.float32),
            pltpu.SemaphoreType.DMA((_NG,)),
            pltpu.SemaphoreType.DMA((_NO,)),
        ],
    )(xt, lut2)
    return out_t.transpose(2, 0, 1)


# speedup vs baseline: 1.3242x; 1.2943x over previous
import math

import jax
import jax.numpy as jnp
from jax import lax
from jax.experimental import pallas as pl
from jax.experimental.pallas import tpu as pltpu
from jax.experimental.pallas import tpu_sc as plsc

_V = 1000000
_D = 64
_B = 4096
_H = 200
_NC = 2
_NS = 16
_NW = _NC * _NS
_BB = _B // _NW
_NG = 4
_NO = 2
_SCALE = math.sqrt(_D)
_L = 16
_W = 4096


def _relayout_body(a_ref, out_ref):
    r = lax.broadcasted_iota(jnp.int32, (_D, _D), 0)
    c = lax.broadcasted_iota(jnp.int32, (_D, _D), 1)
    eye = jnp.where(r == c, _SCALE, 0.0).astype(jnp.float32)
    out_ref[...] = lax.dot_general(
        a_ref[...], eye, (((0,), (0,)), ((), ())),
        preferred_element_type=jnp.float32)


def _gather_body(xt_hbm, lut_hbm, out_hbm, idx_all, gbuf, obuf, gsem, ssem):
    wid = lax.axis_index("s") * _NC + lax.axis_index("c")
    b0 = wid * _BB
    lane = lax.iota(jnp.int32, _L)

    pltpu.sync_copy(xt_hbm.at[:, pl.ds(b0, _BB)], idx_all)

    def start_gather(s, h):
        pltpu.async_copy(lut_hbm.at[idx_all.at[h]], gbuf.at[s], gsem.at[s])

    def wait_gather(s):
        pltpu.make_async_copy(lut_hbm.at[idx_all.at[0]], gbuf.at[s],
                              gsem.at[s]).wait()

    def start_store(so, h):
        pltpu.async_copy(obuf.at[so], out_hbm.at[h, :, pl.ds(b0, _BB)],
                         ssem.at[so])

    def wait_store(so):
        pltpu.make_async_copy(obuf.at[so], out_hbm.at[0, :, pl.ds(b0, _BB)],
                              ssem.at[so]).wait()

    for s in range(_NG):
        start_gather(s, s)

    def outer(g, carry):
        for s in range(_NG):
            h = g * _NG + s
            so = s % _NO
            wait_gather(s)

            @pl.when(h >= _NO)
            def _():
                wait_store(so)

            src = gbuf.at[s]
            dst = obuf.at[so]

            @plsc.parallel_loop(0, _L, step=1, unroll=4)
            def _(d):
                diag = (lane + d) & (_L - 1)
                for db in range(_D // _L):
                    for jb in range(_BB // _L):
                        vals = plsc.load_gather(
                            src, [lane + jb * _L, diag + db * _L])
                        plsc.store_scatter(
                            dst, [diag + db * _L, lane + jb * _L], vals)

            start_store(so, h)

            @pl.when(g < (_H // _NG) - 1)
            def _():
                start_gather(s, h + _NG)
        return carry

    lax.fori_loop(0, _H // _NG, outer, 0)

    for so in range(_NO):
        wait_store(so)


def kernel(x, lut):
    xt = x.T.astype(jnp.int32)
    lutt = lut.T
    lut_rm = pl.pallas_call(
        _relayout_body,
        grid=(pl.cdiv(_V, _W),),
        in_specs=[pl.BlockSpec((_D, _W), lambda i: (0, i))],
        out_specs=pl.BlockSpec((_W, _D), lambda i: (i, 0)),
        out_shape=jax.ShapeDtypeStruct((_V, _D), jnp.float32),
    )(lutt)
    mesh = plsc.VectorSubcoreMesh(
        core_axis_name="c", subcore_axis_name="s",
        num_cores=_NC, num_subcores=_NS,
    )
    params = pltpu.CompilerParams(
        use_tc_tiling_on_sc=False, needs_layout_passes=False,
    )
    out_t = pl.kernel(
        _gather_body,
        out_type=jax.ShapeDtypeStruct((_H, _D, _B), jnp.float32),
        mesh=mesh,
        compiler_params=params,
        scratch_types=[
            pltpu.VMEM((_H, _BB), jnp.int32),
            pltpu.VMEM((_NG, _BB, _D), jnp.float32),
            pltpu.VMEM((_NO, _D, _BB), jnp.float32),
            pltpu.SemaphoreType.DMA((_NG,)),
            pltpu.SemaphoreType.DMA((_NO,)),
        ],
    )(xt, lut_rm)
    return out_t.transpose(2, 0, 1)


# speedup vs baseline: 3.1127x; 2.3506x over previous
import math

import jax
import jax.numpy as jnp
from jax import lax
from jax.experimental import pallas as pl
from jax.experimental.pallas import tpu as pltpu
from jax.experimental.pallas import tpu_sc as plsc

_V = 1000000
_F = 499712
_VP = _V - _F
_D = 64
_B = 4096
_H = 200
_NC = 2
_NS = 16
_NW = _NC * _NS
_BB = _B // _NW
_NG = 4
_NO = 2
_SCALE = math.sqrt(_D)
_L = 16
_W = 8192


def _relayout_body(a_ref, b_ref, out_ref):
    r = lax.broadcasted_iota(jnp.int32, (_D, _D), 0)
    c = lax.broadcasted_iota(jnp.int32, (_D, _D), 1)
    eye = jnp.where(r == c, _SCALE, 0.0).astype(jnp.float32)
    dims = (((0,), (0,)), ((), ()))
    out_ref[:, :_D] = lax.dot_general(
        a_ref[...], eye, dims, preferred_element_type=jnp.float32)
    out_ref[:, _D:] = lax.dot_general(
        b_ref[...], eye, dims, preferred_element_type=jnp.float32)


def _gather_body(xt_hbm, lut2_hbm, out_hbm, idx_all, pidx, poff, gbuf, obuf,
                 gsem, ssem):
    wid = lax.axis_index("s") * _NC + lax.axis_index("c")
    b0 = wid * _BB
    lane = lax.iota(jnp.int32, _L)

    pltpu.sync_copy(xt_hbm.at[:, pl.ds(b0, _BB)], idx_all)

    def prep_and_gather(s, h):
        for j in range(_BB // _L):
            sl = pl.ds(j * _L, _L)
            v = idx_all[h, sl]
            big = v >= _F
            pidx[s, sl] = jnp.where(big, v - _F, v)
            poff[s, sl] = jnp.where(big, _D, 0)
        pltpu.async_copy(lut2_hbm.at[pidx.at[s]], gbuf.at[s], gsem.at[s])

    def wait_gather(s):
        pltpu.make_async_copy(lut2_hbm.at[pidx.at[s]], gbuf.at[s],
                              gsem.at[s]).wait()

    def start_store(so, h):
        pltpu.async_copy(obuf.at[so], out_hbm.at[h, :, pl.ds(b0, _BB)],
                         ssem.at[so])

    def wait_store(so):
        pltpu.make_async_copy(obuf.at[so], out_hbm.at[0, :, pl.ds(b0, _BB)],
                              ssem.at[so]).wait()

    for s in range(_NG):
        prep_and_gather(s, s)

    def outer(g, carry):
        for s in range(_NG):
            h = g * _NG + s
            so = s % _NO
            wait_gather(s)

            @pl.when(h >= _NO)
            def _():
                wait_store(so)

            src = gbuf.at[s]
            dst = obuf.at[so]
            offs = [poff[s, pl.ds(jb * _L, _L)] for jb in range(_BB // _L)]

            @plsc.parallel_loop(0, _L, step=1, unroll=4)
            def _(d):
                diag = (lane + d) & (_L - 1)
                for db in range(_D // _L):
                    for jb in range(_BB // _L):
                        vals = plsc.load_gather(
                            src,
                            [lane + jb * _L, offs[jb] + (diag + db * _L)])
                        plsc.store_scatter(
                            dst, [diag + db * _L, lane + jb * _L], vals)

            start_store(so, h)

            @pl.when(g < (_H // _NG) - 1)
            def _():
                prep_and_gather(s, h + _NG)
        return carry

    lax.fori_loop(0, _H // _NG, outer, 0)

    for so in range(_NO):
        wait_store(so)


def kernel(x, lut):
    xt = x.T.astype(jnp.int32)
    lutt = lut.T
    nblk = pl.cdiv(_VP, _W)
    lut2 = pl.pallas_call(
        _relayout_body,
        grid=(nblk,),
        in_specs=[
            pl.BlockSpec((_D, _W), lambda i: (0, i)),
            pl.BlockSpec((_D, _W), lambda i: (0, _F // _W + i)),
        ],
        out_specs=pl.BlockSpec((_W, 2 * _D), lambda i: (i, 0)),
        out_shape=jax.ShapeDtypeStruct((_VP, 2 * _D), jnp.float32),
    )(lutt, lutt)
    mesh = plsc.VectorSubcoreMesh(
        core_axis_name="c", subcore_axis_name="s",
        num_cores=_NC, num_subcores=_NS,
    )
    params = pltpu.CompilerParams(
        use_tc_tiling_on_sc=True, needs_layout_passes=False,
    )
    out_t = pl.kernel(
        _gather_body,
        out_type=jax.ShapeDtypeStruct((_H, _D, _B), jnp.float32),
        mesh=mesh,
        compiler_params=params,
        scratch_types=[
            pltpu.VMEM((_H, _BB), jnp.int32),
            pltpu.VMEM((_NG, _BB), jnp.int32),
            pltpu.VMEM((_NG, _BB), jnp.int32),
            pltpu.VMEM((_NG, _BB, 2 * _D), jnp.float32),
            pltpu.VMEM((_NO, _D, _BB), jnp.float32),
            pltpu.SemaphoreType.DMA((_NG,)),
            pltpu.SemaphoreType.DMA((_NO,)),
        ],
    )(xt, lut2)
    return out_t.transpose(2, 0, 1)
